# baseline (speedup 1.0000x reference)
import functools

import jax
import jax.numpy as jnp
from jax import lax
from jax.experimental import pallas as pl
from jax.experimental.pallas import tpu as pltpu
from jax.experimental.pallas import tpu_sc as plsc

BATCH = 16384
EMBED = 32
FEAT = 2 * EMBED
ROWS_PACKED = 4
VROWS = 1000000 // ROWS_PACKED
NW = 32
CHUNK = 128
B_PER_W = BATCH // NW
NCH = B_PER_W // CHUNK
BN_EPS = 1e-5

_mesh = plsc.VectorSubcoreMesh(core_axis_name="c", subcore_axis_name="s")


@functools.partial(
    pl.kernel,
    mesh=_mesh,
    compiler_params=pltpu.CompilerParams(needs_layout_passes=False),
    out_type=jax.ShapeDtypeStruct((BATCH, EMBED), jnp.float32),
    scratch_types=[
        pltpu.VMEM((NCH, CHUNK), jnp.int32),
        pltpu.VMEM((NCH, CHUNK), jnp.int32),
        pltpu.VMEM((2, CHUNK, ROWS_PACKED * EMBED), jnp.float32),
        pltpu.VMEM((CHUNK, EMBED), jnp.float32),
        pltpu.SemaphoreType.DMA,
    ],
)
def _gather_one(t_hbm, q_hbm, tab_hbm, out_hbm, t_v, q_v, g_v, rb_v, sem):
    c = lax.axis_index("c")
    s = lax.axis_index("s")
    wid = s * 2 + c
    pltpu.sync_copy(t_hbm.at[wid], t_v)
    pltpu.sync_copy(q_hbm.at[wid], q_v)

    def fire(j, b):
        pltpu.async_copy(tab_hbm.at[t_v.at[j]], g_v.at[b], sem)

    def drain(b):
        pltpu.make_async_copy(tab_hbm.at[t_v.at[0]], g_v.at[b], sem).wait()

    iota16 = lax.iota(jnp.int32, 16)
    fire(0, 0)
    for j in range(NCH):
        b = j % 2
        drain(b)
        if j + 1 < NCH:
            fire(j + 1, (j + 1) % 2)
        def extract(g, carry, j=j, b=b):
            qv = q_v[j, pl.ds(g * 16, 16)] * EMBED
            for t in range(16):
                k = g * 16 + t
                ks = jnp.full((16,), k, jnp.int32)
                for h in range(EMBED // 16):
                    off = h * 16 + iota16
                    v = plsc.load_gather(g_v.at[b], [ks, qv[t] + off])
                    rb_v[k, pl.ds(h * 16, 16)] = v
            return carry

        lax.fori_loop(0, CHUNK // 16, extract, None)
        row0 = (wid * NCH + j) * CHUNK
        pltpu.sync_copy(rb_v, out_hbm.at[pl.ds(row0, CHUNK)])


BM = 2048


def _mlp_body(xu_ref, xi_ref, gu_ref, gi_ref, beu_ref, bei_ref,
              muu_ref, mui_ref, vau_ref, vai_ref,
              W1u_ref, W1i_ref, b1_ref, W2_ref, b2_ref, W3_ref, b3_ref,
              Wo_ref, bo_ref, o_ref):
    su = gu_ref[...] * lax.rsqrt(vau_ref[...] + BN_EPS)
    si = gi_ref[...] * lax.rsqrt(vai_ref[...] + BN_EPS)
    xu = xu_ref[...] * su + (beu_ref[...] - muu_ref[...] * su)
    xi = xi_ref[...] * si + (bei_ref[...] - mui_ref[...] * si)
    h = (jnp.dot(xu, W1u_ref[...], preferred_element_type=jnp.float32)
         + jnp.dot(xi, W1i_ref[...], preferred_element_type=jnp.float32)
         + b1_ref[...])
    h = jnp.maximum(h, 0.0)
    h = jnp.maximum(jnp.dot(h, W2_ref[...], preferred_element_type=jnp.float32) + b2_ref[...], 0.0)
    h = jnp.maximum(jnp.dot(h, W3_ref[...], preferred_element_type=jnp.float32) + b3_ref[...], 0.0)
    o_ref[...] = jnp.dot(h, Wo_ref[...], preferred_element_type=jnp.float32) + bo_ref[...]


def _full(shape):
    return pl.BlockSpec(shape, lambda i: tuple(0 for _ in shape))


_mlp = pl.pallas_call(
    _mlp_body,
    grid=(BATCH // BM,),
    in_specs=[
        pl.BlockSpec((BM, EMBED), lambda i: (i, 0)),
        pl.BlockSpec((BM, EMBED), lambda i: (i, 0)),
        _full((1, EMBED)), _full((1, EMBED)), _full((1, EMBED)),
        _full((1, EMBED)), _full((1, EMBED)), _full((1, EMBED)),
        _full((1, EMBED)), _full((1, EMBED)),
        _full((EMBED, 32)), _full((EMBED, 32)), _full((1, 32)),
        _full((32, 16)), _full((1, 16)),
        _full((16, 8)), _full((1, 8)),
        _full((8, 1)), _full((1, 1)),
    ],
    out_specs=pl.BlockSpec((BM, 1), lambda i: (i, 0)),
    out_shape=jax.ShapeDtypeStruct((BATCH, 1), jnp.float32),
)


def kernel(users, items, user_table, movie_table, bn_gamma, bn_beta, bn_mean,
           bn_var, W1, b1, W2, b2, W3, b3, Wo, bo):
    users = users.astype(jnp.int32)
    items = items.astype(jnp.int32)
    ut = (users >> 2).reshape(NW, NCH, CHUNK)
    it = (items >> 2).reshape(NW, NCH, CHUNK)
    uq = (users & 3).reshape(NW, NCH, CHUNK)
    iq = (items & 3).reshape(NW, NCH, CHUNK)
    utab = user_table.reshape(VROWS, ROWS_PACKED * EMBED)
    mtab = movie_table.reshape(VROWS, ROWS_PACKED * EMBED)
    xu = _gather_one(ut, uq, utab)
    xi = _gather_one(it, iq, mtab)
    rating = _mlp(
        xu, xi,
        bn_gamma[:EMBED].reshape(1, EMBED), bn_gamma[EMBED:].reshape(1, EMBED),
        bn_beta[:EMBED].reshape(1, EMBED), bn_beta[EMBED:].reshape(1, EMBED),
        bn_mean[:EMBED].reshape(1, EMBED), bn_mean[EMBED:].reshape(1, EMBED),
        bn_var[:EMBED].reshape(1, EMBED), bn_var[EMBED:].reshape(1, EMBED),
        W1[:EMBED], W1[EMBED:], b1.reshape(1, 32),
        W2, b2.reshape(1, 16),
        W3, b3.reshape(1, 8),
        Wo, bo.reshape(1, 1),
    )
    return rating


# speedup vs baseline: 1.0100x; 1.0100x over previous
import functools

import jax
import jax.numpy as jnp
from jax import lax
from jax.experimental import pallas as pl
from jax.experimental.pallas import tpu as pltpu
from jax.experimental.pallas import tpu_sc as plsc

BATCH = 16384
EMBED = 32
FEAT = 2 * EMBED
NW = 32
CHUNK = 128
B_PER_W = BATCH // NW
NCH = B_PER_W // CHUNK
BN_EPS = 1e-5

_mesh = plsc.VectorSubcoreMesh(core_axis_name="c", subcore_axis_name="s")


@functools.partial(
    pl.kernel,
    mesh=_mesh,
    compiler_params=pltpu.CompilerParams(use_tc_tiling_on_sc=False),
    out_type=jax.ShapeDtypeStruct((NW, NCH, CHUNK, 2, EMBED), jnp.float32),
    scratch_types=[
        pltpu.VMEM((NCH, CHUNK), jnp.int32),
        pltpu.VMEM((NCH, CHUNK), jnp.int32),
        pltpu.VMEM((NCH, CHUNK, EMBED), jnp.float32),
        pltpu.VMEM((NCH, CHUNK, EMBED), jnp.float32),
        pltpu.SemaphoreType.DMA,
    ],
)
def _gather_embeddings(users_hbm, items_hbm, utab_hbm, mtab_hbm, out_hbm,
                       uidx_v, iidx_v, ubuf_v, ibuf_v, sem):
    c = lax.axis_index("c")
    s = lax.axis_index("s")
    wid = s * 2 + c
    pltpu.sync_copy(users_hbm.at[wid], uidx_v)
    pltpu.sync_copy(items_hbm.at[wid], iidx_v)
    copies = []
    for j in range(NCH):
        copies.append(pltpu.async_copy(utab_hbm.at[uidx_v.at[j]], ubuf_v.at[j], sem))
        copies.append(pltpu.async_copy(mtab_hbm.at[iidx_v.at[j]], ibuf_v.at[j], sem))
    for cp in copies:
        cp.wait()
    for j in range(NCH):
        pltpu.sync_copy(ubuf_v.at[j], out_hbm.at[wid, j, :, 0])
        pltpu.sync_copy(ibuf_v.at[j], out_hbm.at[wid, j, :, 1])


BM = 2048


def _mlp_body(x_ref, g_ref, be_ref, mu_ref, var_ref,
              W1_ref, b1_ref, W2_ref, b2_ref, W3_ref, b3_ref, Wo_ref, bo_ref,
              o_ref):
    s = g_ref[...] * lax.rsqrt(var_ref[...] + BN_EPS)
    x = x_ref[...] * s + (be_ref[...] - mu_ref[...] * s)
    h = jnp.maximum(jnp.dot(x, W1_ref[...], preferred_element_type=jnp.float32) + b1_ref[...], 0.0)
    h = jnp.maximum(jnp.dot(h, W2_ref[...], preferred_element_type=jnp.float32) + b2_ref[...], 0.0)
    h = jnp.maximum(jnp.dot(h, W3_ref[...], preferred_element_type=jnp.float32) + b3_ref[...], 0.0)
    o_ref[...] = jnp.dot(h, Wo_ref[...], preferred_element_type=jnp.float32) + bo_ref[...]


def _full(shape):
    return pl.BlockSpec(shape, lambda i: (0, 0))


_mlp = pl.pallas_call(
    _mlp_body,
    grid=(BATCH // BM,),
    in_specs=[
        pl.BlockSpec((BM, FEAT), lambda i: (i, 0)),
        _full((1, FEAT)), _full((1, FEAT)), _full((1, FEAT)), _full((1, FEAT)),
        _full((FEAT, 32)), _full((1, 32)),
        _full((32, 16)), _full((1, 16)),
        _full((16, 8)), _full((1, 8)),
        _full((8, 1)), _full((1, 1)),
    ],
    out_specs=pl.BlockSpec((BM, 1), lambda i: (i, 0)),
    out_shape=jax.ShapeDtypeStruct((BATCH, 1), jnp.float32),
)


def kernel(users, items, user_table, movie_table, bn_gamma, bn_beta, bn_mean,
           bn_var, W1, b1, W2, b2, W3, b3, Wo, bo):
    users_r = users.astype(jnp.int32).reshape(NW, NCH, CHUNK)
    items_r = items.astype(jnp.int32).reshape(NW, NCH, CHUNK)
    x = _gather_embeddings(users_r, items_r, user_table, movie_table)
    x = x.reshape(BATCH, FEAT)
    rating = _mlp(
        x,
        bn_gamma.reshape(1, FEAT), bn_beta.reshape(1, FEAT),
        bn_mean.reshape(1, FEAT), bn_var.reshape(1, FEAT),
        W1, b1.reshape(1, 32),
        W2, b2.reshape(1, 16),
        W3, b3.reshape(1, 8),
        Wo, bo.reshape(1, 1),
    )
    return rating
